# initial kernel scaffold (unmeasured)
import jax
import jax.numpy as jnp
from jax import lax
from jax.experimental import pallas as pl
from jax.experimental.pallas import tpu as pltpu

N_DEV = 4
F8_MAX = 448.0


def kernel(x, w_mat):
    m_per, k = x.shape
    _, n = w_mat.shape
    n_per = n // N_DEV
    m_glob = N_DEV * m_per

    def body(x_ref, w_ref, out_ref, y_ref, amax_ref, q_send_ref, q_recv_ref,
             data_send_sems, data_recv_sems, amax_send_sems, amax_recv_sems):
        my = lax.axis_index("i")

        barrier_sem = pltpu.get_barrier_semaphore()
        for off in range(1, N_DEV):
            pl.semaphore_signal(
                barrier_sem, inc=1,
                device_id=((my + off) % N_DEV,),
                device_id_type=pl.DeviceIdType.MESH,
            )
        pl.semaphore_wait(barrier_sem, N_DEV - 1)

        am = jnp.float32(0.0)
        for j in range(N_DEV):
            yj = jnp.dot(
                x_ref[...],
                w_ref[:, j * n_per:(j + 1) * n_per],
                preferred_element_type=jnp.float32,
            )
            yj = jnp.maximum(yj, 0.0)
            y_ref[j] = yj
            am = jnp.maximum(am, jnp.max(yj))
        amax_ref[0] = am * jnp.ones((8, 128), jnp.float32)

        amax_rdmas = []
        for off in range(1, N_DEV):
            peer = (my + off) % N_DEV
            r = pltpu.make_async_remote_copy(
                src_ref=amax_ref.at[0],
                dst_ref=amax_ref.at[N_DEV - off],
                send_sem=amax_send_sems.at[off],
                recv_sem=amax_recv_sems.at[N_DEV - off],
                device_id=(peer,),
                device_id_type=pl.DeviceIdType.MESH,
            )
            r.start()
            amax_rdmas.append(r)
        for r in amax_rdmas:
            r.wait_recv()
        gmax = jnp.max(amax_ref[...])
        inv_scale = F8_MAX / gmax
        scale = gmax / F8_MAX

        data_rdmas = []
        for off in range(1, N_DEV):
            peer = (my + off) % N_DEV
            chunk = y_ref[peer]
            q = jnp.clip(chunk * inv_scale, 0.0, F8_MAX)
            q_send_ref[off] = q.astype(jnp.float8_e4m3fn)
            r = pltpu.make_async_remote_copy(
                src_ref=q_send_ref.at[off],
                dst_ref=q_recv_ref.at[N_DEV - off],
                send_sem=data_send_sems.at[off],
                recv_sem=data_recv_sems.at[N_DEV - off],
                device_id=(peer,),
                device_id_type=pl.DeviceIdType.MESH,
            )
            r.start()
            data_rdmas.append(r)

        own = y_ref[my]
        q_own = jnp.clip(own * inv_scale, 0.0, F8_MAX).astype(jnp.float8_e4m3fn)
        pl.store(
            out_ref,
            (pl.ds(my * m_per, m_per), slice(None)),
            q_own.astype(jnp.float32) * scale,
        )

        for off, r in zip(range(1, N_DEV), data_rdmas):
            r.wait_recv()
            slot = N_DEV - off
            origin = (my + slot) % N_DEV
            pl.store(
                out_ref,
                (pl.ds(origin * m_per, m_per), slice(None)),
                q_recv_ref[slot].astype(jnp.float32) * scale,
            )

        for r in amax_rdmas + data_rdmas:
            r.wait_send()

    return pl.pallas_call(
        body,
        out_shape=jax.ShapeDtypeStruct((m_glob, n_per), jnp.float32),
        in_specs=[
            pl.BlockSpec(memory_space=pltpu.VMEM),
            pl.BlockSpec(memory_space=pltpu.VMEM),
        ],
        out_specs=pl.BlockSpec(memory_space=pltpu.VMEM),
        scratch_shapes=[
            pltpu.VMEM((N_DEV, m_per, n_per), jnp.float32),
            pltpu.VMEM((N_DEV, 8, 128), jnp.float32),
            pltpu.VMEM((N_DEV, m_per, n_per), jnp.float8_e4m3fn),
            pltpu.VMEM((N_DEV, m_per, n_per), jnp.float8_e4m3fn),
            pltpu.SemaphoreType.DMA((N_DEV,)),
            pltpu.SemaphoreType.DMA((N_DEV,)),
            pltpu.SemaphoreType.DMA((N_DEV,)),
            pltpu.SemaphoreType.DMA((N_DEV,)),
        ],
        compiler_params=pltpu.CompilerParams(collective_id=0),
    )(x, w_mat)


# baseline (device time: 58585 ns/iter reference)
import jax
import jax.numpy as jnp
from jax import lax
from jax.experimental import pallas as pl
from jax.experimental.pallas import tpu as pltpu

N_DEV = 4
F8_MAX = 448.0


def kernel(x, w_mat):
    m_per, k = x.shape
    _, n = w_mat.shape
    n_per = n // N_DEV
    m_glob = N_DEV * m_per

    def body(x_ref, w_hbm_ref, out_ref, w_buf, y_ref, amax_ref,
             q_send_ref, q_recv_ref, w_sems,
             data_send_sems, data_recv_sems, amax_send_sems, amax_recv_sems):
        my = lax.axis_index("i")

        barrier_sem = pltpu.get_barrier_semaphore()
        for off in range(1, N_DEV):
            pl.semaphore_signal(
                barrier_sem, inc=1,
                device_id=((my + off) % N_DEV,),
                device_id_type=pl.DeviceIdType.MESH,
            )
        pl.semaphore_wait(barrier_sem, N_DEV - 1)

        def w_dma(j):
            return pltpu.make_async_copy(
                w_hbm_ref.at[:, pl.ds(j * n_per, n_per)],
                w_buf.at[j % 2],
                w_sems.at[j % 2],
            )

        w_dma(0).start()
        w_dma(1).start()
        am = jnp.float32(0.0)
        for j in range(N_DEV):
            w_dma(j).wait()
            yj = jnp.dot(
                x_ref[...], w_buf[j % 2],
                preferred_element_type=jnp.float32,
            )
            if j + 2 < N_DEV:
                w_dma(j + 2).start()
            yj = jnp.maximum(yj, 0.0)
            y_ref[j] = yj
            am = jnp.maximum(am, jnp.max(yj))
        amax_ref[0] = am * jnp.ones((8, 128), jnp.float32)

        amax_rdmas = []
        for off in range(1, N_DEV):
            peer = (my + off) % N_DEV
            r = pltpu.make_async_remote_copy(
                src_ref=amax_ref.at[0],
                dst_ref=amax_ref.at[N_DEV - off],
                send_sem=amax_send_sems.at[off - 1],
                recv_sem=amax_recv_sems.at[N_DEV - off - 1],
                device_id=(peer,),
                device_id_type=pl.DeviceIdType.MESH,
            )
            r.start()
            amax_rdmas.append(r)
        for r in amax_rdmas:
            r.wait_recv()
        gmax = jnp.max(amax_ref[...])
        inv_scale = F8_MAX / gmax
        scale = gmax / F8_MAX

        data_rdmas = []
        for off in range(1, N_DEV):
            peer = (my + off) % N_DEV
            chunk = y_ref[peer]
            q = jnp.clip(chunk * inv_scale, 0.0, F8_MAX)
            q_send_ref[off - 1] = q.astype(jnp.float8_e4m3fn)
            r = pltpu.make_async_remote_copy(
                src_ref=q_send_ref.at[off - 1],
                dst_ref=q_recv_ref.at[N_DEV - off - 1],
                send_sem=data_send_sems.at[off - 1],
                recv_sem=data_recv_sems.at[N_DEV - off - 1],
                device_id=(peer,),
                device_id_type=pl.DeviceIdType.MESH,
            )
            r.start()
            data_rdmas.append(r)

        own = y_ref[my]
        q_own = jnp.clip(own * inv_scale, 0.0, F8_MAX).astype(jnp.float8_e4m3fn)
        out_ref[pl.ds(my * m_per, m_per), :] = q_own.astype(jnp.float32) * scale

        for off, r in zip(range(1, N_DEV), data_rdmas):
            r.wait_recv()
            slot = N_DEV - off - 1
            origin = (my + slot + 1) % N_DEV
            out_ref[pl.ds(origin * m_per, m_per), :] = (
                q_recv_ref[slot].astype(jnp.float32) * scale
            )

        for r in amax_rdmas + data_rdmas:
            r.wait_send()

    return pl.pallas_call(
        body,
        out_shape=jax.ShapeDtypeStruct((m_glob, n_per), jnp.float32),
        in_specs=[
            pl.BlockSpec(memory_space=pltpu.VMEM),
            pl.BlockSpec(memory_space=pl.ANY),
        ],
        out_specs=pl.BlockSpec(memory_space=pltpu.VMEM),
        scratch_shapes=[
            pltpu.VMEM((2, k, n_per), jnp.float32),
            pltpu.VMEM((N_DEV, m_per, n_per), jnp.float32),
            pltpu.VMEM((N_DEV, 8, 128), jnp.float32),
            pltpu.VMEM((N_DEV - 1, m_per, n_per), jnp.float8_e4m3fn),
            pltpu.VMEM((N_DEV - 1, m_per, n_per), jnp.float8_e4m3fn),
            pltpu.SemaphoreType.DMA((2,)),
            pltpu.SemaphoreType.DMA((N_DEV - 1,)),
            pltpu.SemaphoreType.DMA((N_DEV - 1,)),
            pltpu.SemaphoreType.DMA((N_DEV - 1,)),
            pltpu.SemaphoreType.DMA((N_DEV - 1,)),
        ],
        compiler_params=pltpu.CompilerParams(
            collective_id=0, vmem_limit_bytes=100 * 1024 * 1024
        ),
    )(x, w_mat)


# device time: 35084 ns/iter; 1.6698x vs baseline; 1.6698x over previous
import os

import jax
import jax.numpy as jnp
from jax import lax
from jax.experimental import pallas as pl
from jax.experimental.pallas import tpu as pltpu

N_DEV = 4
F8_MAX = 448.0

_MODE = os.environ.get("KMODE", "full")


def kernel(x, w_mat):
    m_per, k = x.shape
    _, n = w_mat.shape
    n_per = n // N_DEV
    m_glob = N_DEV * m_per

    def body(x_ref, w_hbm_ref, out_ref, w_buf, y_ref, amax_ref,
             q_send_ref, q_recv_ref, w_sems,
             data_send_sems, data_recv_sems, amax_send_sems, amax_recv_sems):
        my = lax.axis_index("i")

        if _MODE != "gemm":
            with jax.named_scope("barrier"):
                barrier_sem = pltpu.get_barrier_semaphore()
                for off in range(1, N_DEV):
                    pl.semaphore_signal(
                        barrier_sem, inc=1,
                        device_id=((my + off) % N_DEV,),
                        device_id_type=pl.DeviceIdType.MESH,
                    )
                pl.semaphore_wait(barrier_sem, N_DEV - 1)

        def w_dma(j):
            return pltpu.make_async_copy(
                w_hbm_ref.at[:, pl.ds(j * n_per, n_per)],
                w_buf.at[j % 2],
                w_sems.at[j % 2],
            )

        w_dma(0).start()
        w_dma(1).start()
        am = jnp.float32(0.0)
        for j in range(N_DEV):
            with jax.named_scope(f"gemm#j={j}"):
                w_dma(j).wait()
                yj = jnp.dot(
                    x_ref[...], w_buf[j % 2],
                    preferred_element_type=jnp.float32,
                )
                if j + 2 < N_DEV:
                    w_dma(j + 2).start()
                yj = jnp.maximum(yj, 0.0)
                y_ref[j] = yj
                am = jnp.maximum(am, jnp.max(yj))
        amax_ref[0] = am * jnp.ones((8, 128), jnp.float32)

        amax_rdmas = []
        if _MODE == "full":
            with jax.named_scope("amax_exchange"):
                for off in range(1, N_DEV):
                    peer = (my + off) % N_DEV
                    r = pltpu.make_async_remote_copy(
                        src_ref=amax_ref.at[0],
                        dst_ref=amax_ref.at[N_DEV - off],
                        send_sem=amax_send_sems.at[off - 1],
                        recv_sem=amax_recv_sems.at[N_DEV - off - 1],
                        device_id=(peer,),
                        device_id_type=pl.DeviceIdType.MESH,
                    )
                    r.start()
                    amax_rdmas.append(r)
                for r in amax_rdmas:
                    r.wait_recv()
            gmax = jnp.max(amax_ref[...])
        else:
            gmax = jnp.max(amax_ref[0])
        inv_scale = F8_MAX / gmax
        scale = gmax / F8_MAX

        if _MODE == "gemm":
            for j in range(N_DEV):
                q = jnp.clip(y_ref[j] * inv_scale, 0.0, F8_MAX).astype(
                    jnp.float8_e4m3fn
                )
                out_ref[pl.ds(j * m_per, m_per), :] = (
                    q.astype(jnp.float32) * scale
                )
            return

        data_rdmas = []
        for off in range(1, N_DEV):
            with jax.named_scope(f"quant_send#off={off}"):
                peer = (my + off) % N_DEV
                chunk = y_ref[peer]
                q = jnp.clip(chunk * inv_scale, 0.0, F8_MAX)
                q_send_ref[off - 1] = q.astype(jnp.float8_e4m3fn)
                r = pltpu.make_async_remote_copy(
                    src_ref=q_send_ref.at[off - 1],
                    dst_ref=q_recv_ref.at[N_DEV - off - 1],
                    send_sem=data_send_sems.at[off - 1],
                    recv_sem=data_recv_sems.at[N_DEV - off - 1],
                    device_id=(peer,),
                    device_id_type=pl.DeviceIdType.MESH,
                )
                r.start()
                data_rdmas.append(r)

        with jax.named_scope("own_chunk"):
            own = y_ref[my]
            q_own = jnp.clip(own * inv_scale, 0.0, F8_MAX).astype(
                jnp.float8_e4m3fn
            )
            out_ref[pl.ds(my * m_per, m_per), :] = (
                q_own.astype(jnp.float32) * scale
            )

        for off, r in zip(range(1, N_DEV), data_rdmas):
            with jax.named_scope(f"recv_store#off={off}"):
                r.wait_recv()
                slot = N_DEV - off - 1
                origin = (my + slot + 1) % N_DEV
                out_ref[pl.ds(origin * m_per, m_per), :] = (
                    q_recv_ref[slot].astype(jnp.float32) * scale
                )

        with jax.named_scope("drain"):
            for r in amax_rdmas + data_rdmas:
                r.wait_send()

    return pl.pallas_call(
        body,
        out_shape=jax.ShapeDtypeStruct((m_glob, n_per), jnp.float32),
        in_specs=[
            pl.BlockSpec(memory_space=pltpu.VMEM),
            pl.BlockSpec(memory_space=pl.ANY),
        ],
        out_specs=pl.BlockSpec(memory_space=pltpu.VMEM),
        scratch_shapes=[
            pltpu.VMEM((2, k, n_per), jnp.float32),
            pltpu.VMEM((N_DEV, m_per, n_per), jnp.float32),
            pltpu.VMEM((N_DEV, 8, 128), jnp.float32),
            pltpu.VMEM((N_DEV - 1, m_per, n_per), jnp.float8_e4m3fn),
            pltpu.VMEM((N_DEV - 1, m_per, n_per), jnp.float8_e4m3fn),
            pltpu.SemaphoreType.DMA((2,)),
            pltpu.SemaphoreType.DMA((N_DEV - 1,)),
            pltpu.SemaphoreType.DMA((N_DEV - 1,)),
            pltpu.SemaphoreType.DMA((N_DEV - 1,)),
            pltpu.SemaphoreType.DMA((N_DEV - 1,)),
        ],
        compiler_params=pltpu.CompilerParams(
            collective_id=None if _MODE == "gemm" else 0,
            vmem_limit_bytes=100 * 1024 * 1024,
        ),
    )(x, w_mat)
